# Initial kernel scaffold; baseline (speedup 1.0000x reference)
import jax
import jax.numpy as jnp
from jax.experimental import pallas as pl


def kernel(input):
    raise NotImplementedError("write your pallas kernel here")


# speedup vs baseline: 19.9812x; 19.9812x over previous
import jax
import jax.numpy as jnp
from jax.experimental import pallas as pl

_N_ITERS = 22
_BLOCK_ROWS = 256


def _sparsemax_block(x_ref, o_ref):
    x = x_ref[...]
    mx = jnp.max(x, axis=1, keepdims=True)
    lo = mx - 1.0
    hi = mx

    def body(_, carry):
        lo, hi = carry
        mid = 0.5 * (lo + hi)
        f = jnp.sum(jnp.maximum(x - mid, 0.0), axis=1, keepdims=True)
        gt = f > 1.0
        lo = jnp.where(gt, mid, lo)
        hi = jnp.where(gt, hi, mid)
        return lo, hi

    lo, hi = jax.lax.fori_loop(0, _N_ITERS, body, (lo, hi))
    tau = 0.5 * (lo + hi)
    o_ref[...] = jnp.maximum(x - tau, 0.0)


def kernel(input):
    orig_shape = input.shape
    n = orig_shape[-1]
    x2 = input.reshape(-1, n)
    rows = x2.shape[0]
    out = pl.pallas_call(
        _sparsemax_block,
        grid=(rows // _BLOCK_ROWS,),
        in_specs=[pl.BlockSpec((_BLOCK_ROWS, n), lambda i: (i, 0))],
        out_specs=pl.BlockSpec((_BLOCK_ROWS, n), lambda i: (i, 0)),
        out_shape=jax.ShapeDtypeStruct((rows, n), x2.dtype),
    )(x2)
    return out.reshape(orig_shape)


# speedup vs baseline: 27.9221x; 1.3974x over previous
import jax
import jax.numpy as jnp
from jax.experimental import pallas as pl

_N_ITERS = 14
_BLOCK_ROWS = 256


def _sparsemax_block(x_ref, o_ref):
    x = x_ref[...]
    mx = jnp.max(x, axis=1, keepdims=True)
    lo = mx - 1.0
    hi = mx

    def body(_, carry):
        lo, hi = carry
        mid = 0.5 * (lo + hi)
        f = jnp.sum(jnp.maximum(x - mid, 0.0), axis=1, keepdims=True)
        gt = f > 1.0
        lo = jnp.where(gt, mid, lo)
        hi = jnp.where(gt, hi, mid)
        return lo, hi

    lo, hi = jax.lax.fori_loop(0, _N_ITERS, body, (lo, hi))
    mask = x > lo
    s = jnp.sum(jnp.where(mask, x, 0.0), axis=1, keepdims=True)
    k = jnp.sum(mask.astype(x.dtype), axis=1, keepdims=True)
    tau = jnp.clip((s - 1.0) / k, lo, hi)
    o_ref[...] = jnp.maximum(x - tau, 0.0)


def kernel(input):
    orig_shape = input.shape
    n = orig_shape[-1]
    x2 = input.reshape(-1, n)
    rows = x2.shape[0]
    out = pl.pallas_call(
        _sparsemax_block,
        grid=(rows // _BLOCK_ROWS,),
        in_specs=[pl.BlockSpec((_BLOCK_ROWS, n), lambda i: (i, 0))],
        out_specs=pl.BlockSpec((_BLOCK_ROWS, n), lambda i: (i, 0)),
        out_shape=jax.ShapeDtypeStruct((rows, n), x2.dtype),
    )(x2)
    return out.reshape(orig_shape)
